# Initial kernel scaffold; baseline (speedup 1.0000x reference)
import jax
import jax.numpy as jnp
from jax.experimental import pallas as pl


def kernel(X, ei_feat, batch, W_nuc, b_nuc, W_surf, b_surf, W_fuse, b_fuse):
    raise NotImplementedError("write your pallas kernel here")


# speedup vs baseline: 26.7899x; 26.7899x over previous
import functools

import jax
import jax.numpy as jnp
from jax import lax
from jax.experimental import pallas as pl
from jax.experimental.pallas import tpu as pltpu
from jax.experimental.pallas import tpu_sc as plsc

N = 10000
E = 320000
NC, NS = 2, 16
NW = NC * NS
CHUNK = 128
CPW = -(-E // (NW * CHUNK))
E_PAD = NW * CHUNK * CPW
ZROWS = 632
N_PAD = NS * ZROWS
ZH = ZROWS // 2

_mesh = plsc.VectorSubcoreMesh(
    core_axis_name="c", subcore_axis_name="s", num_cores=NC, num_subcores=NS)


def _zero_fill(buf, rows, cols):
    def body(i, _):
        for k in range(cols // 16):
            buf[i, pl.ds(k * 16, 16)] = jnp.zeros((16,), jnp.float32)
        return 0
    lax.fori_loop(0, rows, body, 0)


def _deg_body(dst_hbm, out_hbm, dstv, ones_v, zer_v, acc):
    c = lax.axis_index("c")
    s = lax.axis_index("s")
    w = c * NS + s
    def ones_fill(i, _):
        ones_v[i, pl.ds(0, 16)] = jnp.ones((16,), jnp.float32)
        return 0
    lax.fori_loop(0, CHUNK, ones_fill, 0)
    _zero_fill(zer_v, ZH, 16)
    pltpu.sync_copy(dst_hbm.at[w], dstv)
    pltpu.sync_copy(zer_v, acc.at[pl.ds(s * ZROWS, ZH)])
    pltpu.sync_copy(zer_v, acc.at[pl.ds(s * ZROWS + ZH, ZH)])
    plsc.subcore_barrier()
    def step(j, _):
        pltpu.sync_copy(ones_v, acc.at[dstv.at[j]], add=True)
        return 0
    lax.fori_loop(0, CPW, step, 0)
    plsc.subcore_barrier()
    pltpu.sync_copy(acc.at[pl.ds(s * ZROWS, ZROWS)],
                    out_hbm.at[c, pl.ds(s * ZROWS, ZROWS)])


_deg_call = pl.kernel(
    _deg_body,
    out_type=jax.ShapeDtypeStruct((NC, N_PAD, 16), jnp.float32),
    mesh=_mesh,
    scratch_types=[
        pltpu.VMEM((CPW, CHUNK), jnp.int32),
        pltpu.VMEM((CHUNK, 16), jnp.float32),
        pltpu.VMEM((ZH, 16), jnp.float32),
        pltpu.VMEM_SHARED((N_PAD, 16), jnp.float32),
    ],
    compiler_params=pltpu.CompilerParams(use_tc_tiling_on_sc=False),
)


def _prop_body(width, tab_hbm, src_hbm, dst_hbm, out_hbm,
               srcv, dstv, rows, zer_v, acc, gsem):
    c = lax.axis_index("c")
    s = lax.axis_index("s")
    w = c * NS + s
    _zero_fill(zer_v, ZH, width)
    pltpu.sync_copy(src_hbm.at[w], srcv)
    pltpu.sync_copy(dst_hbm.at[w], dstv)
    pltpu.sync_copy(zer_v, acc.at[pl.ds(s * ZROWS, ZH)])
    pltpu.sync_copy(zer_v, acc.at[pl.ds(s * ZROWS + ZH, ZH)])
    plsc.subcore_barrier()
    def step(j, _):
        pltpu.async_copy(tab_hbm.at[srcv.at[j]], rows, gsem).wait()
        pltpu.sync_copy(rows, acc.at[dstv.at[j]], add=True)
        return 0
    lax.fori_loop(0, CPW, step, 0)
    plsc.subcore_barrier()
    pltpu.sync_copy(acc.at[pl.ds(s * ZROWS, ZROWS)],
                    out_hbm.at[c, pl.ds(s * ZROWS, ZROWS)])


def _make_prop(width):
    return pl.kernel(
        functools.partial(_prop_body, width),
        out_type=jax.ShapeDtypeStruct((NC, N_PAD, width), jnp.float32),
        mesh=_mesh,
        scratch_types=[
            pltpu.VMEM((CPW, CHUNK), jnp.int32),
            pltpu.VMEM((CPW, CHUNK), jnp.int32),
            pltpu.VMEM((CHUNK, width), jnp.float32),
            pltpu.VMEM((ZH, width), jnp.float32),
            pltpu.VMEM_SHARED((N_PAD, width), jnp.float32),
            pltpu.SemaphoreType.DMA,
        ],
        compiler_params=pltpu.CompilerParams(use_tc_tiling_on_sc=False),
    )


_prop64 = _make_prop(64)
_prop16 = _make_prop(16)


def _tc_a_body(x_ref, wn_ref, ws_ref, degp_ref, m1_ref, dinv_ref):
    degp = degp_ref[0] + degp_ref[1]
    deg = 1.0 + degp[:N, :1]
    dinv = lax.rsqrt(deg)
    x = x_ref[...]
    h1 = jnp.concatenate(
        [jnp.dot(x[:, :64], wn_ref[...], preferred_element_type=jnp.float32,
                 precision=lax.Precision.HIGHEST),
         jnp.dot(x[:, 64:], ws_ref[...], preferred_element_type=jnp.float32,
                 precision=lax.Precision.HIGHEST)],
        axis=1)
    m1_ref[...] = h1 * dinv
    dinv_ref[...] = dinv


_tc_a = pl.pallas_call(
    _tc_a_body,
    out_shape=[jax.ShapeDtypeStruct((N, 64), jnp.float32),
               jax.ShapeDtypeStruct((N, 1), jnp.float32)],
)


def _tc_b_body(p1_ref, m1_ref, dinv_ref, bcat_ref, wf_ref, m2_ref):
    dinv = dinv_ref[...]
    p1 = (p1_ref[0] + p1_ref[1])[:N]
    pre = (p1 + m1_ref[...]) * dinv + bcat_ref[...]
    h = jnp.where(pre > 0, pre, jnp.exp(jnp.minimum(pre, 0.0)) - 1.0)
    m2_ref[...] = jnp.dot(h, wf_ref[...], preferred_element_type=jnp.float32,
                          precision=lax.Precision.HIGHEST) * dinv


_tc_b = pl.pallas_call(
    _tc_b_body,
    out_shape=jax.ShapeDtypeStruct((N, 16), jnp.float32),
)


def _tc_c_body(p2_ref, m2_ref, dinv_ref, bf_ref, out_ref):
    p2 = (p2_ref[0] + p2_ref[1])[:N]
    logits = (p2 + m2_ref[...]) * dinv_ref[...] + bf_ref[...]
    mx = jnp.max(logits, axis=1, keepdims=True)
    e = jnp.exp(logits - mx)
    out_ref[...] = e / jnp.sum(e, axis=1, keepdims=True)


_tc_c = pl.pallas_call(
    _tc_c_body,
    out_shape=jax.ShapeDtypeStruct((N, 16), jnp.float32),
)


def kernel(X, ei_feat, batch, W_nuc, b_nuc, W_surf, b_surf, W_fuse, b_fuse):
    src = ei_feat[0]
    dst = ei_feat[1]
    pad = E_PAD - E
    srcp = jnp.concatenate(
        [src, jnp.zeros((pad,), jnp.int32)]).reshape(NW, CPW, CHUNK)
    dstp = jnp.concatenate(
        [dst, jnp.full((pad,), N, jnp.int32)]).reshape(NW, CPW, CHUNK)

    degp = _deg_call(dstp)
    m1, dinv = _tc_a(X, W_nuc, W_surf, degp)
    p1 = _prop64(m1, srcp, dstp)
    bcat = jnp.concatenate([b_nuc, b_surf])[None, :]
    m2 = _tc_b(p1, m1, dinv, bcat, W_fuse)
    p2 = _prop16(m2, srcp, dstp)
    return _tc_c(p2, m2, dinv, b_fuse[None, :])


# speedup vs baseline: 35.1844x; 1.3133x over previous
import functools

import jax
import jax.numpy as jnp
from jax import lax
from jax.experimental import pallas as pl
from jax.experimental.pallas import tpu as pltpu
from jax.experimental.pallas import tpu_sc as plsc

N = 10000
E = 320000
NC, NS = 2, 16
NW = NC * NS
CHUNK = 128
CPW = -(-E // (NW * CHUNK))
NBUF = 4
E_PAD = NW * CHUNK * CPW
ZROWS = 632
N_PAD = NS * ZROWS
ZH = ZROWS // 2

_mesh = plsc.VectorSubcoreMesh(
    core_axis_name="c", subcore_axis_name="s", num_cores=NC, num_subcores=NS)


def _zero_fill(buf, rows, cols):
    def body(i, _):
        for k in range(cols // 16):
            buf[i, pl.ds(k * 16, 16)] = jnp.zeros((16,), jnp.float32)
        return 0
    lax.fori_loop(0, rows, body, 0)


def _deg_body(dst_hbm, out_hbm, dstv, ones_v, zer_v, acc):
    c = lax.axis_index("c")
    s = lax.axis_index("s")
    w = c * NS + s
    def ones_fill(i, _):
        ones_v[i, pl.ds(0, 16)] = jnp.ones((16,), jnp.float32)
        return 0
    lax.fori_loop(0, CHUNK, ones_fill, 0)
    _zero_fill(zer_v, ZH, 16)
    pltpu.sync_copy(dst_hbm.at[w], dstv)
    pltpu.sync_copy(zer_v, acc.at[pl.ds(s * ZROWS, ZH)])
    pltpu.sync_copy(zer_v, acc.at[pl.ds(s * ZROWS + ZH, ZH)])
    plsc.subcore_barrier()
    def step(j, _):
        pltpu.sync_copy(ones_v, acc.at[dstv.at[j]], add=True)
        return 0
    lax.fori_loop(0, CPW, step, 0)
    plsc.subcore_barrier()
    pltpu.sync_copy(acc.at[pl.ds(s * ZROWS, ZROWS)],
                    out_hbm.at[c, pl.ds(s * ZROWS, ZROWS)])


_deg_call = pl.kernel(
    _deg_body,
    out_type=jax.ShapeDtypeStruct((NC, N_PAD, 16), jnp.float32),
    mesh=_mesh,
    scratch_types=[
        pltpu.VMEM((CPW, CHUNK), jnp.int32),
        pltpu.VMEM((CHUNK, 16), jnp.float32),
        pltpu.VMEM((ZH, 16), jnp.float32),
        pltpu.VMEM_SHARED((N_PAD, 16), jnp.float32),
    ],
    compiler_params=pltpu.CompilerParams(use_tc_tiling_on_sc=False),
)


def _prop_body(width, tab_hbm, src_hbm, dst_hbm, out_hbm,
               srcv, dstv, rows, zer_v, acc, gsem):
    c = lax.axis_index("c")
    s = lax.axis_index("s")
    w = c * NS + s
    _zero_fill(zer_v, ZH, width)
    pltpu.sync_copy(src_hbm.at[w], srcv)
    pltpu.sync_copy(dst_hbm.at[w], dstv)
    pltpu.sync_copy(zer_v, acc.at[pl.ds(s * ZROWS, ZH)])
    pltpu.sync_copy(zer_v, acc.at[pl.ds(s * ZROWS + ZH, ZH)])
    plsc.subcore_barrier()
    for b in range(NBUF - 1):
        pltpu.async_copy(tab_hbm.at[srcv.at[b]], rows.at[b], gsem.at[b])
    def step(j, _):
        slot = lax.rem(j, NBUF)
        nj = j + NBUF - 1
        @pl.when(nj < CPW)
        def _():
            nslot = lax.rem(nj, NBUF)
            pltpu.async_copy(tab_hbm.at[srcv.at[nj]], rows.at[nslot],
                             gsem.at[nslot])
        pltpu.make_async_copy(tab_hbm.at[srcv.at[j]], rows.at[slot],
                              gsem.at[slot]).wait()
        pltpu.sync_copy(rows.at[slot], acc.at[dstv.at[j]], add=True)
        return 0
    lax.fori_loop(0, CPW, step, 0)
    plsc.subcore_barrier()
    pltpu.sync_copy(acc.at[pl.ds(s * ZROWS, ZROWS)],
                    out_hbm.at[c, pl.ds(s * ZROWS, ZROWS)])


def _make_prop(width):
    return pl.kernel(
        functools.partial(_prop_body, width),
        out_type=jax.ShapeDtypeStruct((NC, N_PAD, width), jnp.float32),
        mesh=_mesh,
        scratch_types=[
            pltpu.VMEM((CPW, CHUNK), jnp.int32),
            pltpu.VMEM((CPW, CHUNK), jnp.int32),
            pltpu.VMEM((NBUF, CHUNK, width), jnp.float32),
            pltpu.VMEM((ZH, width), jnp.float32),
            pltpu.VMEM_SHARED((N_PAD, width), jnp.float32),
            pltpu.SemaphoreType.DMA((NBUF,)),
        ],
        compiler_params=pltpu.CompilerParams(use_tc_tiling_on_sc=False),
    )


_prop64 = _make_prop(64)
_prop16 = _make_prop(16)


def _tc_a_body(x_ref, wn_ref, ws_ref, degp_ref, m1_ref, dinv_ref):
    degp = degp_ref[0] + degp_ref[1]
    deg = 1.0 + degp[:N, :1]
    dinv = lax.rsqrt(deg)
    x = x_ref[...]
    h1 = jnp.concatenate(
        [jnp.dot(x[:, :64], wn_ref[...], preferred_element_type=jnp.float32,
                 precision=lax.Precision.HIGHEST),
         jnp.dot(x[:, 64:], ws_ref[...], preferred_element_type=jnp.float32,
                 precision=lax.Precision.HIGHEST)],
        axis=1)
    m1_ref[...] = h1 * dinv
    dinv_ref[...] = dinv


_tc_a = pl.pallas_call(
    _tc_a_body,
    out_shape=[jax.ShapeDtypeStruct((N, 64), jnp.float32),
               jax.ShapeDtypeStruct((N, 1), jnp.float32)],
)


def _tc_b_body(p1_ref, m1_ref, dinv_ref, bcat_ref, wf_ref, m2_ref):
    dinv = dinv_ref[...]
    p1 = (p1_ref[0] + p1_ref[1])[:N]
    pre = (p1 + m1_ref[...]) * dinv + bcat_ref[...]
    h = jnp.where(pre > 0, pre, jnp.exp(jnp.minimum(pre, 0.0)) - 1.0)
    m2_ref[...] = jnp.dot(h, wf_ref[...], preferred_element_type=jnp.float32,
                          precision=lax.Precision.HIGHEST) * dinv


_tc_b = pl.pallas_call(
    _tc_b_body,
    out_shape=jax.ShapeDtypeStruct((N, 16), jnp.float32),
)


def _tc_c_body(p2_ref, m2_ref, dinv_ref, bf_ref, out_ref):
    p2 = (p2_ref[0] + p2_ref[1])[:N]
    logits = (p2 + m2_ref[...]) * dinv_ref[...] + bf_ref[...]
    mx = jnp.max(logits, axis=1, keepdims=True)
    e = jnp.exp(logits - mx)
    out_ref[...] = e / jnp.sum(e, axis=1, keepdims=True)


_tc_c = pl.pallas_call(
    _tc_c_body,
    out_shape=jax.ShapeDtypeStruct((N, 16), jnp.float32),
)


def kernel(X, ei_feat, batch, W_nuc, b_nuc, W_surf, b_surf, W_fuse, b_fuse):
    src = ei_feat[0]
    dst = ei_feat[1]
    pad = E_PAD - E
    srcp = jnp.concatenate(
        [src, jnp.zeros((pad,), jnp.int32)]).reshape(NW, CPW, CHUNK)
    dstp = jnp.concatenate(
        [dst, jnp.full((pad,), N, jnp.int32)]).reshape(NW, CPW, CHUNK)

    degp = _deg_call(dstp)
    m1, dinv = _tc_a(X, W_nuc, W_surf, degp)
    p1 = _prop64(m1, srcp, dstp)
    bcat = jnp.concatenate([b_nuc, b_surf])[None, :]
    m2 = _tc_b(p1, m1, dinv, bcat, W_fuse)
    p2 = _prop16(m2, srcp, dstp)
    return _tc_c(p2, m2, dinv, b_fuse[None, :])


# speedup vs baseline: 47.1210x; 1.3393x over previous
import functools

import jax
import jax.numpy as jnp
from jax import lax
from jax.experimental import pallas as pl
from jax.experimental.pallas import tpu as pltpu
from jax.experimental.pallas import tpu_sc as plsc

N = 10000
E = 320000
NC, NS = 2, 16
NW = NC * NS
CHUNK = 128
CPW = -(-E // (NW * CHUNK))
NBUF = 3
E_PAD = NW * CHUNK * CPW
ZROWS = 632
N_PAD = NS * ZROWS
ZH = ZROWS // 2

_mesh = plsc.VectorSubcoreMesh(
    core_axis_name="c", subcore_axis_name="s", num_cores=NC, num_subcores=NS)


def _zero_fill(buf, rows, cols):
    def body(i, _):
        for k in range(cols // 16):
            buf[i, pl.ds(k * 16, 16)] = jnp.zeros((16,), jnp.float32)
        return 0
    lax.fori_loop(0, rows, body, 0)


def _deg_body(dst_hbm, out_hbm, dstv, ones_v, zer_v, acc):
    c = lax.axis_index("c")
    s = lax.axis_index("s")
    w = c * NS + s
    def ones_fill(i, _):
        ones_v[i, pl.ds(0, 16)] = jnp.ones((16,), jnp.float32)
        return 0
    lax.fori_loop(0, CHUNK, ones_fill, 0)
    _zero_fill(zer_v, ZH, 16)
    pltpu.sync_copy(dst_hbm.at[w], dstv)
    pltpu.sync_copy(zer_v, acc.at[pl.ds(s * ZROWS, ZH)])
    pltpu.sync_copy(zer_v, acc.at[pl.ds(s * ZROWS + ZH, ZH)])
    plsc.subcore_barrier()
    def step(j, _):
        pltpu.sync_copy(ones_v, acc.at[dstv.at[j]], add=True)
        return 0
    lax.fori_loop(0, CPW, step, 0)
    plsc.subcore_barrier()
    pltpu.sync_copy(acc.at[pl.ds(s * ZROWS, ZROWS)],
                    out_hbm.at[c, pl.ds(s * ZROWS, ZROWS)])


_deg_call = pl.kernel(
    _deg_body,
    out_type=jax.ShapeDtypeStruct((NC, N_PAD, 16), jnp.float32),
    mesh=_mesh,
    scratch_types=[
        pltpu.VMEM((CPW, CHUNK), jnp.int32),
        pltpu.VMEM((CHUNK, 16), jnp.float32),
        pltpu.VMEM((ZH, 16), jnp.float32),
        pltpu.VMEM_SHARED((N_PAD, 16), jnp.float32),
    ],
    compiler_params=pltpu.CompilerParams(use_tc_tiling_on_sc=False),
)


def _prop_body(width, tab_hbm, src_hbm, dst_hbm, out_hbm,
               srcv, dstv, rows, tabsp, acc, gsem):
    c = lax.axis_index("c")
    s = lax.axis_index("s")
    w = c * NS + s
    _zero_fill(rows.at[0], CHUNK, width)
    pltpu.sync_copy(src_hbm.at[w], srcv)
    pltpu.sync_copy(dst_hbm.at[w], dstv)
    pltpu.sync_copy(tab_hbm.at[pl.ds(s * (N // NS), N // NS)],
                    tabsp.at[pl.ds(s * (N // NS), N // NS)])
    for k in range(ZROWS // CHUNK):
        pltpu.sync_copy(rows.at[0], acc.at[pl.ds(s * ZROWS + k * CHUNK, CHUNK)])
    _zr = ZROWS % CHUNK
    pltpu.sync_copy(rows.at[0, pl.ds(0, _zr)],
                    acc.at[pl.ds(s * ZROWS + (ZROWS // CHUNK) * CHUNK, _zr)])
    plsc.subcore_barrier()
    for b in range(NBUF - 1):
        pltpu.async_copy(tabsp.at[srcv.at[b]], rows.at[b], gsem.at[b])
    def step(j, _):
        slot = lax.rem(j, NBUF)
        nj = j + NBUF - 1
        @pl.when(nj < CPW)
        def _():
            nslot = lax.rem(nj, NBUF)
            pltpu.async_copy(tabsp.at[srcv.at[nj]], rows.at[nslot],
                             gsem.at[nslot])
        pltpu.make_async_copy(tabsp.at[srcv.at[j]], rows.at[slot],
                              gsem.at[slot]).wait()
        pltpu.sync_copy(rows.at[slot], acc.at[dstv.at[j]], add=True)
        return 0
    lax.fori_loop(0, CPW, step, 0)
    plsc.subcore_barrier()
    pltpu.sync_copy(acc.at[pl.ds(s * ZROWS, ZROWS)],
                    out_hbm.at[c, pl.ds(s * ZROWS, ZROWS)])


def _make_prop(width):
    return pl.kernel(
        functools.partial(_prop_body, width),
        out_type=jax.ShapeDtypeStruct((NC, N_PAD, width), jnp.float32),
        mesh=_mesh,
        scratch_types=[
            pltpu.VMEM((CPW, CHUNK), jnp.int32),
            pltpu.VMEM((CPW, CHUNK), jnp.int32),
            pltpu.VMEM((NBUF, CHUNK, width), jnp.float32),
            pltpu.VMEM_SHARED((N, width), jnp.float32),
            pltpu.VMEM_SHARED((N_PAD, width), jnp.float32),
            pltpu.SemaphoreType.DMA((NBUF,)),
        ],
        compiler_params=pltpu.CompilerParams(use_tc_tiling_on_sc=False),
    )


_prop64 = _make_prop(64)
_prop16 = _make_prop(16)


BR = N_PAD // 8


def _tc_a_body(x_ref, wn_ref, ws_ref, degp_ref, m1_ref, dinv_ref):
    deg = 1.0 + degp_ref[0, :, :1] + degp_ref[1, :, :1]
    dinv = lax.rsqrt(deg)
    x = x_ref[...]
    h1 = jnp.concatenate(
        [jnp.dot(x[:, :64], wn_ref[...], preferred_element_type=jnp.float32,
                 precision=lax.Precision.HIGHEST),
         jnp.dot(x[:, 64:], ws_ref[...], preferred_element_type=jnp.float32,
                 precision=lax.Precision.HIGHEST)],
        axis=1)
    m1_ref[...] = h1 * dinv
    dinv_ref[...] = dinv


_tc_a = pl.pallas_call(
    _tc_a_body,
    grid=(8,),
    in_specs=[
        pl.BlockSpec((BR, 128), lambda i: (i, 0)),
        pl.BlockSpec((64, 32), lambda i: (0, 0)),
        pl.BlockSpec((64, 32), lambda i: (0, 0)),
        pl.BlockSpec((2, BR, 16), lambda i: (0, i, 0)),
    ],
    out_specs=[pl.BlockSpec((BR, 64), lambda i: (i, 0)),
               pl.BlockSpec((BR, 1), lambda i: (i, 0))],
    out_shape=[jax.ShapeDtypeStruct((N, 64), jnp.float32),
               jax.ShapeDtypeStruct((N, 1), jnp.float32)],
)


def _tc_b_body(p1_ref, m1_ref, dinv_ref, bcat_ref, wf_ref, m2_ref):
    dinv = dinv_ref[...]
    p1 = p1_ref[0] + p1_ref[1]
    pre = (p1 + m1_ref[...]) * dinv + bcat_ref[...]
    h = jnp.where(pre > 0, pre, jnp.exp(jnp.minimum(pre, 0.0)) - 1.0)
    m2_ref[...] = jnp.dot(h, wf_ref[...], preferred_element_type=jnp.float32,
                          precision=lax.Precision.HIGHEST) * dinv


_tc_b = pl.pallas_call(
    _tc_b_body,
    grid=(8,),
    in_specs=[
        pl.BlockSpec((2, BR, 64), lambda i: (0, i, 0)),
        pl.BlockSpec((BR, 64), lambda i: (i, 0)),
        pl.BlockSpec((BR, 1), lambda i: (i, 0)),
        pl.BlockSpec((1, 64), lambda i: (0, 0)),
        pl.BlockSpec((64, 16), lambda i: (0, 0)),
    ],
    out_specs=pl.BlockSpec((BR, 16), lambda i: (i, 0)),
    out_shape=jax.ShapeDtypeStruct((N, 16), jnp.float32),
)


def _tc_c_body(p2_ref, m2_ref, dinv_ref, bf_ref, out_ref):
    p2 = p2_ref[0] + p2_ref[1]
    logits = (p2 + m2_ref[...]) * dinv_ref[...] + bf_ref[...]
    mx = jnp.max(logits, axis=1, keepdims=True)
    e = jnp.exp(logits - mx)
    out_ref[...] = e / jnp.sum(e, axis=1, keepdims=True)


_tc_c = pl.pallas_call(
    _tc_c_body,
    grid=(8,),
    in_specs=[
        pl.BlockSpec((2, BR, 16), lambda i: (0, i, 0)),
        pl.BlockSpec((BR, 16), lambda i: (i, 0)),
        pl.BlockSpec((BR, 1), lambda i: (i, 0)),
        pl.BlockSpec((1, 16), lambda i: (0, 0)),
    ],
    out_specs=pl.BlockSpec((BR, 16), lambda i: (i, 0)),
    out_shape=jax.ShapeDtypeStruct((N, 16), jnp.float32),
)


def kernel(X, ei_feat, batch, W_nuc, b_nuc, W_surf, b_surf, W_fuse, b_fuse):
    src = ei_feat[0]
    dst = ei_feat[1]
    pad = E_PAD - E
    srcp = jnp.concatenate(
        [src, jnp.zeros((pad,), jnp.int32)]).reshape(NW, CPW, CHUNK)
    dstp = jnp.concatenate(
        [dst, jnp.full((pad,), N, jnp.int32)]).reshape(NW, CPW, CHUNK)

    degp = _deg_call(dstp)
    m1, dinv = _tc_a(X, W_nuc, W_surf, degp)
    p1 = _prop64(m1, srcp, dstp)
    bcat = jnp.concatenate([b_nuc, b_surf])[None, :]
    m2 = _tc_b(p1, m1, dinv, bcat, W_fuse)
    p2 = _prop16(m2, srcp, dstp)
    return _tc_c(p2, m2, dinv, b_fuse[None, :])


# speedup vs baseline: 50.8861x; 1.0799x over previous
import functools

import jax
import jax.numpy as jnp
from jax import lax
from jax.experimental import pallas as pl
from jax.experimental.pallas import tpu as pltpu
from jax.experimental.pallas import tpu_sc as plsc

N = 10000
E = 320000
NC, NS = 2, 16
NW = NC * NS
CHUNK = 128
TOT = E // CHUNK
CPW = -(-TOT // NW)
LAST = TOT - (NW - 1) * CPW
NBUF = 3
ZROWS = 640
N_PAD = NS * ZROWS
BR = N_PAD // 8

_mesh = plsc.VectorSubcoreMesh(
    core_axis_name="c", subcore_axis_name="s", num_cores=NC, num_subcores=NS)


def _zero_fill(buf, rows, cols):
    def body(i, _):
        for k in range(cols // 16):
            buf[i, pl.ds(k * 16, 16)] = jnp.zeros((16,), jnp.float32)
        return 0
    lax.fori_loop(0, rows, body, 0)


def _stage_indices(ei_hbm, row, w, dstv):
    @pl.when(w < NW - 1)
    def _():
        pltpu.sync_copy(ei_hbm.at[row, pl.ds(w * CPW, CPW)], dstv)

    @pl.when(w == NW - 1)
    def _():
        pltpu.sync_copy(ei_hbm.at[row, pl.ds(w * CPW, LAST)],
                        dstv.at[pl.ds(0, LAST)])


def _deg_body(ei_hbm, out_hbm, dstv, ones_v, acc):
    c = lax.axis_index("c")
    s = lax.axis_index("s")
    w = c * NS + s
    nw = jnp.where(w == NW - 1, LAST, CPW)
    _stage_indices(ei_hbm, 1, w, dstv)
    _zero_fill(ones_v, CHUNK, 16)
    for k in range(ZROWS // CHUNK):
        pltpu.sync_copy(ones_v, acc.at[pl.ds(s * ZROWS + k * CHUNK, CHUNK)])
    def ofill(i, _):
        ones_v[i, pl.ds(0, 16)] = jnp.ones((16,), jnp.float32)
        return 0
    lax.fori_loop(0, CHUNK, ofill, 0)
    plsc.subcore_barrier()
    def step(j, _):
        pltpu.sync_copy(ones_v, acc.at[dstv.at[j]], add=True)
        return 0
    lax.fori_loop(0, nw, step, 0)
    plsc.subcore_barrier()
    pltpu.sync_copy(acc.at[pl.ds(s * ZROWS, ZROWS)],
                    out_hbm.at[c, pl.ds(s * ZROWS, ZROWS)])


_deg_call = pl.kernel(
    _deg_body,
    out_type=jax.ShapeDtypeStruct((NC, N_PAD, 16), jnp.float32),
    mesh=_mesh,
    scratch_types=[
        pltpu.VMEM((CPW, CHUNK), jnp.int32),
        pltpu.VMEM((CHUNK, 16), jnp.float32),
        pltpu.VMEM_SHARED((N_PAD, 16), jnp.float32),
    ],
    compiler_params=pltpu.CompilerParams(use_tc_tiling_on_sc=False),
)


def _prop_body(width, tab_hbm, ei_hbm, out_hbm, srcv, dstv, rows, tabsp,
               acc, gsem):
    c = lax.axis_index("c")
    s = lax.axis_index("s")
    w = c * NS + s
    nw = jnp.where(w == NW - 1, LAST, CPW)
    _zero_fill(rows.at[0], CHUNK, width)
    _stage_indices(ei_hbm, 0, w, srcv)
    _stage_indices(ei_hbm, 1, w, dstv)
    pltpu.sync_copy(tab_hbm.at[pl.ds(s * (N // NS), N // NS)],
                    tabsp.at[pl.ds(s * (N // NS), N // NS)])
    for k in range(ZROWS // CHUNK):
        pltpu.sync_copy(rows.at[0], acc.at[pl.ds(s * ZROWS + k * CHUNK, CHUNK)])
    plsc.subcore_barrier()
    for b in range(NBUF - 1):
        pltpu.async_copy(tabsp.at[srcv.at[b]], rows.at[b], gsem.at[b])
    def step(j, _):
        slot = lax.rem(j, NBUF)
        nj = j + NBUF - 1
        @pl.when(nj < nw)
        def _():
            nslot = lax.rem(nj, NBUF)
            pltpu.async_copy(tabsp.at[srcv.at[nj]], rows.at[nslot],
                             gsem.at[nslot])
        pltpu.make_async_copy(tabsp.at[srcv.at[j]], rows.at[slot],
                              gsem.at[slot]).wait()
        pltpu.sync_copy(rows.at[slot], acc.at[dstv.at[j]], add=True)
        return 0
    lax.fori_loop(0, nw, step, 0)
    plsc.subcore_barrier()
    pltpu.sync_copy(acc.at[pl.ds(s * ZROWS, ZROWS)],
                    out_hbm.at[c, pl.ds(s * ZROWS, ZROWS)])


def _make_prop(width):
    return pl.kernel(
        functools.partial(_prop_body, width),
        out_type=jax.ShapeDtypeStruct((NC, N_PAD, width), jnp.float32),
        mesh=_mesh,
        scratch_types=[
            pltpu.VMEM((CPW, CHUNK), jnp.int32),
            pltpu.VMEM((CPW, CHUNK), jnp.int32),
            pltpu.VMEM((NBUF, CHUNK, width), jnp.float32),
            pltpu.VMEM_SHARED((N, width), jnp.float32),
            pltpu.VMEM_SHARED((N_PAD, width), jnp.float32),
            pltpu.SemaphoreType.DMA((NBUF,)),
        ],
        compiler_params=pltpu.CompilerParams(use_tc_tiling_on_sc=False),
    )


_prop64 = _make_prop(64)
_prop16 = _make_prop(16)


def _tc_mm_body(x_ref, wn_ref, ws_ref, h1_ref):
    x = x_ref[...]
    h1_ref[...] = jnp.concatenate(
        [jnp.dot(x[:, :64], wn_ref[...], preferred_element_type=jnp.float32,
                 precision=lax.Precision.HIGHEST),
         jnp.dot(x[:, 64:], ws_ref[...], preferred_element_type=jnp.float32,
                 precision=lax.Precision.HIGHEST)],
        axis=1)


_tc_mm = pl.pallas_call(
    _tc_mm_body,
    grid=(8,),
    in_specs=[
        pl.BlockSpec((BR, 128), lambda i: (i, 0)),
        pl.BlockSpec((64, 32), lambda i: (0, 0)),
        pl.BlockSpec((64, 32), lambda i: (0, 0)),
    ],
    out_specs=pl.BlockSpec((BR, 64), lambda i: (i, 0)),
    out_shape=jax.ShapeDtypeStruct((N, 64), jnp.float32),
)


def _tc_scale_body(h1_ref, degp_ref, m1_ref, dinv_ref):
    deg = 1.0 + degp_ref[0, :, :1] + degp_ref[1, :, :1]
    dinv = lax.rsqrt(deg)
    m1_ref[...] = h1_ref[...] * dinv
    dinv_ref[...] = dinv


_tc_scale = pl.pallas_call(
    _tc_scale_body,
    grid=(8,),
    in_specs=[
        pl.BlockSpec((BR, 64), lambda i: (i, 0)),
        pl.BlockSpec((2, BR, 16), lambda i: (0, i, 0)),
    ],
    out_specs=[pl.BlockSpec((BR, 64), lambda i: (i, 0)),
               pl.BlockSpec((BR, 1), lambda i: (i, 0))],
    out_shape=[jax.ShapeDtypeStruct((N, 64), jnp.float32),
               jax.ShapeDtypeStruct((N, 1), jnp.float32)],
)


def _tc_b_body(p1_ref, m1_ref, dinv_ref, bcat_ref, wf_ref, m2_ref):
    dinv = dinv_ref[...]
    p1 = p1_ref[0] + p1_ref[1]
    pre = (p1 + m1_ref[...]) * dinv + bcat_ref[...]
    h = jnp.where(pre > 0, pre, jnp.exp(jnp.minimum(pre, 0.0)) - 1.0)
    m2_ref[...] = jnp.dot(h, wf_ref[...], preferred_element_type=jnp.float32,
                          precision=lax.Precision.HIGHEST) * dinv


_tc_b = pl.pallas_call(
    _tc_b_body,
    grid=(8,),
    in_specs=[
        pl.BlockSpec((2, BR, 64), lambda i: (0, i, 0)),
        pl.BlockSpec((BR, 64), lambda i: (i, 0)),
        pl.BlockSpec((BR, 1), lambda i: (i, 0)),
        pl.BlockSpec((1, 64), lambda i: (0, 0)),
        pl.BlockSpec((64, 16), lambda i: (0, 0)),
    ],
    out_specs=pl.BlockSpec((BR, 16), lambda i: (i, 0)),
    out_shape=jax.ShapeDtypeStruct((N, 16), jnp.float32),
)


def _tc_c_body(p2_ref, m2_ref, dinv_ref, bf_ref, out_ref):
    p2 = p2_ref[0] + p2_ref[1]
    logits = (p2 + m2_ref[...]) * dinv_ref[...] + bf_ref[...]
    mx = jnp.max(logits, axis=1, keepdims=True)
    e = jnp.exp(logits - mx)
    out_ref[...] = e / jnp.sum(e, axis=1, keepdims=True)


_tc_c = pl.pallas_call(
    _tc_c_body,
    grid=(8,),
    in_specs=[
        pl.BlockSpec((2, BR, 16), lambda i: (0, i, 0)),
        pl.BlockSpec((BR, 16), lambda i: (i, 0)),
        pl.BlockSpec((BR, 1), lambda i: (i, 0)),
        pl.BlockSpec((1, 16), lambda i: (0, 0)),
    ],
    out_specs=pl.BlockSpec((BR, 16), lambda i: (i, 0)),
    out_shape=jax.ShapeDtypeStruct((N, 16), jnp.float32),
)


def kernel(X, ei_feat, batch, W_nuc, b_nuc, W_surf, b_surf, W_fuse, b_fuse):
    eic = ei_feat.reshape(2, TOT, CHUNK)

    degp = _deg_call(eic)
    h1 = _tc_mm(X, W_nuc, W_surf)
    m1, dinv = _tc_scale(h1, degp)
    p1 = _prop64(m1, eic)
    bcat = jnp.concatenate([b_nuc, b_surf])[None, :]
    m2 = _tc_b(p1, m1, dinv, bcat, W_fuse)
    p2 = _prop16(m2, eic)
    return _tc_c(p2, m2, dinv, b_fuse[None, :])


# speedup vs baseline: 50.9291x; 1.0008x over previous
import functools

import jax
import jax.numpy as jnp
from jax import lax
from jax.experimental import pallas as pl
from jax.experimental.pallas import tpu as pltpu
from jax.experimental.pallas import tpu_sc as plsc

N = 10000
E = 320000
NC, NS = 2, 16
NW = NC * NS
CHUNK = 128
TOT = E // CHUNK
CPW = -(-TOT // NW)
LAST = TOT - (NW - 1) * CPW
NBUF = 3
ZROWS = 640
N_PAD = NS * ZROWS
BR = N_PAD // 8

_mesh = plsc.VectorSubcoreMesh(
    core_axis_name="c", subcore_axis_name="s", num_cores=NC, num_subcores=NS)


def _zero_fill(buf, rows, cols):
    def body(i, _):
        for k in range(cols // 16):
            buf[i, pl.ds(k * 16, 16)] = jnp.zeros((16,), jnp.float32)
        return 0
    lax.fori_loop(0, rows, body, 0)


def _stage_indices(ei_hbm, row, w, dstv):
    @pl.when(w < NW - 1)
    def _():
        pltpu.sync_copy(ei_hbm.at[row, pl.ds(w * CPW, CPW)], dstv)

    @pl.when(w == NW - 1)
    def _():
        pltpu.sync_copy(ei_hbm.at[row, pl.ds(w * CPW, LAST)],
                        dstv.at[pl.ds(0, LAST)])


def _deg_body(ei_hbm, out_hbm, dstv, ones_v, acc):
    c = lax.axis_index("c")
    s = lax.axis_index("s")
    w = c * NS + s
    nw = jnp.where(w == NW - 1, LAST, CPW)
    _stage_indices(ei_hbm, 1, w, dstv)
    _zero_fill(ones_v, CHUNK, 16)
    for k in range(ZROWS // CHUNK):
        pltpu.sync_copy(ones_v, acc.at[pl.ds(s * ZROWS + k * CHUNK, CHUNK)])
    def ofill(i, _):
        ones_v[i, pl.ds(0, 16)] = jnp.ones((16,), jnp.float32)
        return 0
    lax.fori_loop(0, CHUNK, ofill, 0)
    plsc.subcore_barrier()
    def step(j, _):
        pltpu.sync_copy(ones_v, acc.at[dstv.at[j]], add=True)
        return 0
    lax.fori_loop(0, nw, step, 0)
    plsc.subcore_barrier()
    pltpu.sync_copy(acc.at[pl.ds(s * ZROWS, ZROWS)],
                    out_hbm.at[c, pl.ds(s * ZROWS, ZROWS)])


_deg_call = pl.kernel(
    _deg_body,
    out_type=jax.ShapeDtypeStruct((NC, N_PAD, 16), jnp.float32),
    mesh=_mesh,
    scratch_types=[
        pltpu.VMEM((CPW, CHUNK), jnp.int32),
        pltpu.VMEM((CHUNK, 16), jnp.float32),
        pltpu.VMEM_SHARED((N_PAD, 16), jnp.float32),
    ],
    compiler_params=pltpu.CompilerParams(use_tc_tiling_on_sc=False),
)


def _prop_body(width, tab_hbm, ei_hbm, out_hbm, srcv, dstv, rows, tabsp,
               acc, gsem, ssem):
    c = lax.axis_index("c")
    s = lax.axis_index("s")
    w = c * NS + s
    nw = jnp.where(w == NW - 1, LAST, CPW)
    _zero_fill(rows.at[0], CHUNK, width)
    _stage_indices(ei_hbm, 0, w, srcv)
    _stage_indices(ei_hbm, 1, w, dstv)
    pltpu.sync_copy(tab_hbm.at[pl.ds(s * (N // NS), N // NS)],
                    tabsp.at[pl.ds(s * (N // NS), N // NS)])
    for k in range(ZROWS // CHUNK):
        pltpu.sync_copy(rows.at[0], acc.at[pl.ds(s * ZROWS + k * CHUNK, CHUNK)])
    plsc.subcore_barrier()
    for b in range(NBUF - 1):
        pltpu.async_copy(tabsp.at[srcv.at[b]], rows.at[b], gsem.at[b])
    def step(j, _):
        slot = lax.rem(j, NBUF)
        nj = j + NBUF - 1
        @pl.when(nj < nw)
        def _():
            nslot = lax.rem(nj, NBUF)
            @pl.when(j > 0)
            def _():
                pltpu.make_async_copy(rows.at[nslot],
                                      acc.at[dstv.at[j - 1]],
                                      ssem.at[nslot]).wait()
            pltpu.async_copy(tabsp.at[srcv.at[nj]], rows.at[nslot],
                             gsem.at[nslot])
        pltpu.make_async_copy(tabsp.at[srcv.at[j]], rows.at[slot],
                              gsem.at[slot]).wait()
        pltpu.async_copy(rows.at[slot], acc.at[dstv.at[j]], ssem.at[slot],
                         add=True)
        return 0
    lax.fori_loop(0, nw, step, 0)
    for b in range(NBUF):
        pltpu.make_async_copy(rows.at[b], acc.at[dstv.at[0]],
                              ssem.at[b]).wait()
    plsc.subcore_barrier()
    pltpu.sync_copy(acc.at[pl.ds(s * ZROWS, ZROWS)],
                    out_hbm.at[c, pl.ds(s * ZROWS, ZROWS)])


def _make_prop(width):
    return pl.kernel(
        functools.partial(_prop_body, width),
        out_type=jax.ShapeDtypeStruct((NC, N_PAD, width), jnp.float32),
        mesh=_mesh,
        scratch_types=[
            pltpu.VMEM((CPW, CHUNK), jnp.int32),
            pltpu.VMEM((CPW, CHUNK), jnp.int32),
            pltpu.VMEM((NBUF, CHUNK, width), jnp.float32),
            pltpu.VMEM_SHARED((N, width), jnp.float32),
            pltpu.VMEM_SHARED((N_PAD, width), jnp.float32),
            pltpu.SemaphoreType.DMA((NBUF,)),
            pltpu.SemaphoreType.DMA((NBUF,)),
        ],
        compiler_params=pltpu.CompilerParams(use_tc_tiling_on_sc=False),
    )


_prop64 = _make_prop(64)
_prop16 = _make_prop(16)


def _tc_mm_body(x_ref, wn_ref, ws_ref, h1_ref):
    x = x_ref[...]
    h1_ref[...] = jnp.concatenate(
        [jnp.dot(x[:, :64], wn_ref[...], preferred_element_type=jnp.float32,
                 precision=lax.Precision.HIGHEST),
         jnp.dot(x[:, 64:], ws_ref[...], preferred_element_type=jnp.float32,
                 precision=lax.Precision.HIGHEST)],
        axis=1)


_tc_mm = pl.pallas_call(
    _tc_mm_body,
    grid=(8,),
    in_specs=[
        pl.BlockSpec((BR, 128), lambda i: (i, 0)),
        pl.BlockSpec((64, 32), lambda i: (0, 0)),
        pl.BlockSpec((64, 32), lambda i: (0, 0)),
    ],
    out_specs=pl.BlockSpec((BR, 64), lambda i: (i, 0)),
    out_shape=jax.ShapeDtypeStruct((N, 64), jnp.float32),
)


def _tc_scale_body(h1_ref, degp_ref, m1_ref, dinv_ref):
    deg = 1.0 + degp_ref[0, :, :1] + degp_ref[1, :, :1]
    dinv = lax.rsqrt(deg)
    m1_ref[...] = h1_ref[...] * dinv
    dinv_ref[...] = dinv


_tc_scale = pl.pallas_call(
    _tc_scale_body,
    grid=(8,),
    in_specs=[
        pl.BlockSpec((BR, 64), lambda i: (i, 0)),
        pl.BlockSpec((2, BR, 16), lambda i: (0, i, 0)),
    ],
    out_specs=[pl.BlockSpec((BR, 64), lambda i: (i, 0)),
               pl.BlockSpec((BR, 1), lambda i: (i, 0))],
    out_shape=[jax.ShapeDtypeStruct((N, 64), jnp.float32),
               jax.ShapeDtypeStruct((N, 1), jnp.float32)],
)


def _tc_b_body(p1_ref, m1_ref, dinv_ref, bcat_ref, wf_ref, m2_ref):
    dinv = dinv_ref[...]
    p1 = p1_ref[0] + p1_ref[1]
    pre = (p1 + m1_ref[...]) * dinv + bcat_ref[...]
    h = jnp.where(pre > 0, pre, jnp.exp(jnp.minimum(pre, 0.0)) - 1.0)
    m2_ref[...] = jnp.dot(h, wf_ref[...], preferred_element_type=jnp.float32,
                          precision=lax.Precision.HIGHEST) * dinv


_tc_b = pl.pallas_call(
    _tc_b_body,
    grid=(8,),
    in_specs=[
        pl.BlockSpec((2, BR, 64), lambda i: (0, i, 0)),
        pl.BlockSpec((BR, 64), lambda i: (i, 0)),
        pl.BlockSpec((BR, 1), lambda i: (i, 0)),
        pl.BlockSpec((1, 64), lambda i: (0, 0)),
        pl.BlockSpec((64, 16), lambda i: (0, 0)),
    ],
    out_specs=pl.BlockSpec((BR, 16), lambda i: (i, 0)),
    out_shape=jax.ShapeDtypeStruct((N, 16), jnp.float32),
)


def _tc_c_body(p2_ref, m2_ref, dinv_ref, bf_ref, out_ref):
    p2 = p2_ref[0] + p2_ref[1]
    logits = (p2 + m2_ref[...]) * dinv_ref[...] + bf_ref[...]
    mx = jnp.max(logits, axis=1, keepdims=True)
    e = jnp.exp(logits - mx)
    out_ref[...] = e / jnp.sum(e, axis=1, keepdims=True)


_tc_c = pl.pallas_call(
    _tc_c_body,
    grid=(8,),
    in_specs=[
        pl.BlockSpec((2, BR, 16), lambda i: (0, i, 0)),
        pl.BlockSpec((BR, 16), lambda i: (i, 0)),
        pl.BlockSpec((BR, 1), lambda i: (i, 0)),
        pl.BlockSpec((1, 16), lambda i: (0, 0)),
    ],
    out_specs=pl.BlockSpec((BR, 16), lambda i: (i, 0)),
    out_shape=jax.ShapeDtypeStruct((N, 16), jnp.float32),
)


def kernel(X, ei_feat, batch, W_nuc, b_nuc, W_surf, b_surf, W_fuse, b_fuse):
    eic = ei_feat.reshape(2, TOT, CHUNK)

    degp = _deg_call(eic)
    h1 = _tc_mm(X, W_nuc, W_surf)
    m1, dinv = _tc_scale(h1, degp)
    p1 = _prop64(m1, eic)
    bcat = jnp.concatenate([b_nuc, b_surf])[None, :]
    m2 = _tc_b(p1, m1, dinv, bcat, W_fuse)
    p2 = _prop16(m2, eic)
    return _tc_c(p2, m2, dinv, b_fuse[None, :])


# speedup vs baseline: 53.7520x; 1.0554x over previous
import functools

import jax
import jax.numpy as jnp
from jax import lax
from jax.experimental import pallas as pl
from jax.experimental.pallas import tpu as pltpu
from jax.experimental.pallas import tpu_sc as plsc

N = 10000
E = 320000
NC, NS = 2, 16
NW = NC * NS
CHUNK = 128
TOT = E // CHUNK
CPW = -(-TOT // NW)
LAST = TOT - (NW - 1) * CPW
NBUF = 3
ZROWS = 640
N_PAD = NS * ZROWS
BR = N_PAD // 8

_mesh = plsc.VectorSubcoreMesh(
    core_axis_name="c", subcore_axis_name="s", num_cores=NC, num_subcores=NS)


def _zero_fill(buf, rows, cols):
    def body(i, _):
        for k in range(cols // 16):
            buf[i, pl.ds(k * 16, 16)] = jnp.zeros((16,), jnp.float32)
        return 0
    lax.fori_loop(0, rows, body, 0)


def _stage_indices(ei_hbm, row, w, dstv):
    @pl.when(w < NW - 1)
    def _():
        pltpu.sync_copy(ei_hbm.at[row, pl.ds(w * CPW, CPW)], dstv)

    @pl.when(w == NW - 1)
    def _():
        pltpu.sync_copy(ei_hbm.at[row, pl.ds(w * CPW, LAST)],
                        dstv.at[pl.ds(0, LAST)])


def _deg_body(ei_hbm, out_hbm, dstv, ones_v, acc):
    c = lax.axis_index("c")
    s = lax.axis_index("s")
    w = c * NS + s
    nw = jnp.where(w == NW - 1, LAST, CPW)
    _stage_indices(ei_hbm, 1, w, dstv)
    _zero_fill(ones_v, CHUNK, 16)
    for k in range(ZROWS // CHUNK):
        pltpu.sync_copy(ones_v, acc.at[pl.ds(s * ZROWS + k * CHUNK, CHUNK)])
    def ofill(i, _):
        ones_v[i, pl.ds(0, 16)] = jnp.ones((16,), jnp.float32)
        return 0
    lax.fori_loop(0, CHUNK, ofill, 0)
    plsc.subcore_barrier()
    def step(j, _):
        pltpu.sync_copy(ones_v, acc.at[dstv.at[j]], add=True)
        return 0
    lax.fori_loop(0, nw, step, 0)
    plsc.subcore_barrier()
    pltpu.sync_copy(acc.at[pl.ds(s * ZROWS, ZROWS)],
                    out_hbm.at[c, pl.ds(s * ZROWS, ZROWS)])


_deg_call = pl.kernel(
    _deg_body,
    out_type=jax.ShapeDtypeStruct((NC, N_PAD, 16), jnp.float32),
    mesh=_mesh,
    scratch_types=[
        pltpu.VMEM((CPW, CHUNK), jnp.int32),
        pltpu.VMEM((CHUNK, 16), jnp.float32),
        pltpu.VMEM_SHARED((N_PAD, 16), jnp.float32),
    ],
    compiler_params=pltpu.CompilerParams(use_tc_tiling_on_sc=False),
)


def _prop_body(width, tab_hbm, ei_hbm, out_hbm, srcv, dstv, rows, tabsp,
               acc, gsem, ssem):
    c = lax.axis_index("c")
    s = lax.axis_index("s")
    w = c * NS + s
    nw = jnp.where(w == NW - 1, LAST, CPW)
    _zero_fill(rows.at[0], CHUNK, width)
    _stage_indices(ei_hbm, 0, w, srcv)
    _stage_indices(ei_hbm, 1, w, dstv)
    pltpu.sync_copy(tab_hbm.at[pl.ds(s * (N // NS), N // NS)],
                    tabsp.at[pl.ds(s * (N // NS), N // NS)])
    for k in range(ZROWS // CHUNK):
        pltpu.sync_copy(rows.at[0], acc.at[pl.ds(s * ZROWS + k * CHUNK, CHUNK)])
    plsc.subcore_barrier()
    for b in range(NBUF - 1):
        pltpu.async_copy(tabsp.at[srcv.at[b]], rows.at[b], gsem.at[b])
    def step(j, _):
        slot = lax.rem(j, NBUF)
        nj = j + NBUF - 1
        @pl.when(nj < nw)
        def _():
            nslot = lax.rem(nj, NBUF)
            @pl.when(j > 0)
            def _():
                pltpu.make_async_copy(rows.at[nslot],
                                      acc.at[dstv.at[j - 1]],
                                      ssem.at[nslot]).wait()
            pltpu.async_copy(tabsp.at[srcv.at[nj]], rows.at[nslot],
                             gsem.at[nslot])
        pltpu.make_async_copy(tabsp.at[srcv.at[j]], rows.at[slot],
                              gsem.at[slot]).wait()
        pltpu.async_copy(rows.at[slot], acc.at[dstv.at[j]], ssem.at[slot],
                         add=True)
        return 0
    lax.fori_loop(0, nw, step, 0)
    for b in range(NBUF):
        pltpu.make_async_copy(rows.at[b], acc.at[dstv.at[0]],
                              ssem.at[b]).wait()
    plsc.subcore_barrier()
    pltpu.sync_copy(acc.at[pl.ds(s * ZROWS, ZROWS)],
                    out_hbm.at[c, pl.ds(s * ZROWS, ZROWS)])


def _make_prop(width):
    return pl.kernel(
        functools.partial(_prop_body, width),
        out_type=jax.ShapeDtypeStruct((NC, N_PAD, width), jnp.float32),
        mesh=_mesh,
        scratch_types=[
            pltpu.VMEM((CPW, CHUNK), jnp.int32),
            pltpu.VMEM((CPW, CHUNK), jnp.int32),
            pltpu.VMEM((NBUF, CHUNK, width), jnp.float32),
            pltpu.VMEM_SHARED((N, width), jnp.float32),
            pltpu.VMEM_SHARED((N_PAD, width), jnp.float32),
            pltpu.SemaphoreType.DMA((NBUF,)),
            pltpu.SemaphoreType.DMA((NBUF,)),
        ],
        compiler_params=pltpu.CompilerParams(use_tc_tiling_on_sc=False),
    )


_prop64 = _make_prop(64)
_prop16 = _make_prop(16)


def _tc_mm_body(x_ref, wn_ref, ws_ref, h1_ref):
    x = x_ref[...]
    h1_ref[...] = jnp.concatenate(
        [jnp.dot(x[:, :64], wn_ref[...], preferred_element_type=jnp.float32,
                 precision=lax.Precision.HIGHEST),
         jnp.dot(x[:, 64:], ws_ref[...], preferred_element_type=jnp.float32,
                 precision=lax.Precision.HIGHEST)],
        axis=1)


_tc_mm = pl.pallas_call(
    _tc_mm_body,
    grid=(8,),
    in_specs=[
        pl.BlockSpec((BR, 128), lambda i: (i, 0)),
        pl.BlockSpec((64, 32), lambda i: (0, 0)),
        pl.BlockSpec((64, 32), lambda i: (0, 0)),
    ],
    out_specs=pl.BlockSpec((BR, 64), lambda i: (i, 0)),
    out_shape=jax.ShapeDtypeStruct((N, 64), jnp.float32),
)


def _tc_scale_body(h1_ref, degp_ref, m1_ref, dinv_ref):
    deg = 1.0 + degp_ref[0, :, :1] + degp_ref[1, :, :1]
    dinv = lax.rsqrt(deg)
    m1_ref[...] = h1_ref[...] * dinv
    i = pl.program_id(0)
    dinv_ref[pl.ds(i * BR, BR)] = dinv[:, 0]


_tc_scale = pl.pallas_call(
    _tc_scale_body,
    grid=(8,),
    in_specs=[
        pl.BlockSpec((BR, 64), lambda i: (i, 0)),
        pl.BlockSpec((2, BR, 16), lambda i: (0, i, 0)),
    ],
    out_specs=[pl.BlockSpec((BR, 64), lambda i: (i, 0)),
               pl.BlockSpec((N_PAD,), lambda i: (0,))],
    out_shape=[jax.ShapeDtypeStruct((N, 64), jnp.float32),
               jax.ShapeDtypeStruct((N_PAD,), jnp.float32)],
)


def _tc_b_body(p1_ref, m1_ref, dinv_ref, bcat_ref, wf_ref, m2_ref):
    i = pl.program_id(0)
    dinv = dinv_ref[pl.ds(i * BR, BR)][:, None]
    p1 = p1_ref[0] + p1_ref[1]
    pre = (p1 + m1_ref[...]) * dinv + bcat_ref[...]
    h = jnp.where(pre > 0, pre, jnp.exp(jnp.minimum(pre, 0.0)) - 1.0)
    m2_ref[...] = jnp.dot(h, wf_ref[...], preferred_element_type=jnp.float32,
                          precision=lax.Precision.HIGHEST) * dinv


_tc_b = pl.pallas_call(
    _tc_b_body,
    grid=(8,),
    in_specs=[
        pl.BlockSpec((2, BR, 64), lambda i: (0, i, 0)),
        pl.BlockSpec((BR, 64), lambda i: (i, 0)),
        pl.BlockSpec((N_PAD,), lambda i: (0,)),
        pl.BlockSpec((1, 64), lambda i: (0, 0)),
        pl.BlockSpec((64, 16), lambda i: (0, 0)),
    ],
    out_specs=pl.BlockSpec((BR, 16), lambda i: (i, 0)),
    out_shape=jax.ShapeDtypeStruct((N, 16), jnp.float32),
)


def _tc_c_body(p2_ref, m2_ref, degp_ref, bf_ref, s_ref, out_ref):
    dinv16 = lax.rsqrt(1.0 + degp_ref[0] + degp_ref[1])
    logits = (p2_ref[0] + p2_ref[1] + m2_ref[...]) * dinv16 + bf_ref[...]
    mx = jnp.max(logits, axis=1, keepdims=True)
    e = jnp.exp(logits - mx)
    den = jnp.dot(e, s_ref[...], preferred_element_type=jnp.float32,
                  precision=lax.Precision.HIGHEST)
    out_ref[...] = e / den


NP8 = N_PAD // 8
NR8 = N * 16 // 128
BR8 = NP8 // 8

_tc_c = pl.pallas_call(
    _tc_c_body,
    grid=(8,),
    in_specs=[
        pl.BlockSpec((2, BR8, 128), lambda i: (0, i, 0)),
        pl.BlockSpec((BR8, 128), lambda i: (i, 0)),
        pl.BlockSpec((2, BR8, 128), lambda i: (0, i, 0)),
        pl.BlockSpec((1, 128), lambda i: (0, 0)),
        pl.BlockSpec((128, 128), lambda i: (0, 0)),
    ],
    out_specs=pl.BlockSpec((BR8, 128), lambda i: (i, 0)),
    out_shape=jax.ShapeDtypeStruct((NR8, 128), jnp.float32),
)


def kernel(X, ei_feat, batch, W_nuc, b_nuc, W_surf, b_surf, W_fuse, b_fuse):
    eic = ei_feat.reshape(2, TOT, CHUNK)

    degp = _deg_call(eic)
    h1 = _tc_mm(X, W_nuc, W_surf)
    m1, dinv = _tc_scale(h1, degp)
    p1 = _prop64(m1, eic)
    bcat = jnp.concatenate([b_nuc, b_surf])[None, :]
    m2 = _tc_b(p1, m1, dinv, bcat, W_fuse)
    p2 = _prop16(m2, eic)
    bias128 = jnp.tile(b_fuse, (8,))[None, :]
    seg = jnp.repeat(jnp.repeat(jnp.eye(8, dtype=jnp.float32), 16, axis=0),
                     16, axis=1)
    out8 = _tc_c(p2.reshape(NC, NP8, 128), m2.reshape(NR8, 128),
                 degp.reshape(NC, NP8, 128), bias128, seg)
    return out8.reshape(N, 16)
